# baseline (speedup 1.0000x reference)
import functools

import jax
import jax.numpy as jnp
from jax import lax
from jax.experimental import pallas as pl
from jax.experimental.pallas import tpu as pltpu
from jax.experimental.pallas import tpu_sc as plsc

NC = 2
NS = 16
NW = NC * NS


def _pick_chunk(per_worker):
    for k in range(128, 7, -8):
        if per_worker % k == 0:
            return k
    return 0


def _sc_degree(edge3, n_nodes):
    _, _, C, K = edge3.shape
    rpt = n_nodes // NS
    mesh = plsc.VectorSubcoreMesh(core_axis_name="c", subcore_axis_name="s")

    @functools.partial(
        pl.kernel,
        out_type=jax.ShapeDtypeStruct((NC, n_nodes, 16), jnp.float32),
        mesh=mesh,
        compiler_params=pltpu.CompilerParams(use_tc_tiling_on_sc=False),
        scratch_types=[
            pltpu.VMEM((C, K), jnp.int32),
            pltpu.VMEM((K, 16), jnp.float32),
            pltpu.VMEM((rpt, 16), jnp.float32),
            pltpu.VMEM_SHARED((n_nodes, 16), jnp.float32),
            pltpu.SemaphoreType.DMA,
        ],
    )
    def deg_kernel(edge_hbm, out_hbm, idx_v, ones_v, buf_v, acc_sh, sem):
        c = lax.axis_index("c")
        s = lax.axis_index("s")
        wid = c * NS + s
        pltpu.sync_copy(edge_hbm.at[1, wid], idx_v)

        one_row = jnp.where(lax.iota(jnp.int32, 16) == 0, 1.0, 0.0)

        def fill_ones(i, _):
            ones_v[i, :] = one_row
            return 0

        lax.fori_loop(0, K, fill_ones, 0)

        zv = jnp.zeros((16,), jnp.float32)

        def fill_zero(i, _):
            buf_v[i, :] = zv
            return 0

        lax.fori_loop(0, rpt, fill_zero, 0)
        row0 = s * rpt
        pltpu.sync_copy(buf_v, acc_sh.at[pl.ds(row0, rpt)])
        plsc.subcore_barrier()

        pltpu.async_copy(ones_v, acc_sh.at[idx_v.at[0]], sem, add=True)

        def chunk(j, _):
            @pl.when(j + 1 < C)
            def _():
                pltpu.async_copy(ones_v, acc_sh.at[idx_v.at[j + 1]], sem,
                                 add=True)

            pltpu.make_async_copy(ones_v, acc_sh.at[idx_v.at[j]], sem).wait()
            return 0

        lax.fori_loop(0, C, chunk, 0)
        plsc.subcore_barrier()
        pltpu.sync_copy(acc_sh.at[pl.ds(row0, rpt)], buf_v)
        pltpu.sync_copy(buf_v, out_hbm.at[c, pl.ds(row0, rpt)])

    return deg_kernel(edge3)


def _sc_aggregate(hs, edge3):
    N, H = hs.shape
    _, _, C, K = edge3.shape
    rpt = N // NS
    mesh = plsc.VectorSubcoreMesh(core_axis_name="c", subcore_axis_name="s")

    @functools.partial(
        pl.kernel,
        out_type=jax.ShapeDtypeStruct((NC, N, H), jnp.float32),
        mesh=mesh,
        compiler_params=pltpu.CompilerParams(use_tc_tiling_on_sc=False),
        scratch_types=[
            pltpu.VMEM((C, K), jnp.int32),
            pltpu.VMEM((C, K), jnp.int32),
            pltpu.VMEM((3, K, H), jnp.float32),
            pltpu.VMEM((rpt, H), jnp.float32),
            pltpu.VMEM_SHARED((N, H), jnp.float32),
            pltpu.SemaphoreType.DMA((3,)),
            pltpu.SemaphoreType.DMA((3,)),
        ],
    )
    def agg_kernel(hs_hbm, edge_hbm, out_hbm, sidx, didx, rows, buf,
                   acc, gsem, ssem):
        c = lax.axis_index("c")
        s = lax.axis_index("s")
        wid = c * NS + s
        pltpu.sync_copy(edge_hbm.at[0, wid], sidx)
        pltpu.sync_copy(edge_hbm.at[1, wid], didx)

        row0 = s * rpt
        pltpu.sync_copy(hs_hbm.at[pl.ds(row0, rpt)], buf)
        pltpu.sync_copy(buf, acc.at[pl.ds(row0, rpt)])
        plsc.subcore_barrier()

        for p in range(2):
            pltpu.async_copy(hs_hbm.at[sidx.at[p]], rows.at[p], gsem.at[p])

        def chunk(j, _):
            b = lax.rem(j, 3)
            pltpu.make_async_copy(hs_hbm.at[sidx.at[j]], rows.at[b],
                                  gsem.at[b]).wait()
            pltpu.async_copy(rows.at[b], acc.at[didx.at[j]], ssem.at[b],
                             add=True)
            nb = lax.rem(j + 2, 3)

            @pl.when(j + 2 < C)
            def _():
                @pl.when(j >= 1)
                def _():
                    pltpu.make_async_copy(rows.at[nb], acc.at[didx.at[j - 1]],
                                          ssem.at[nb]).wait()

                pltpu.async_copy(hs_hbm.at[sidx.at[j + 2]], rows.at[nb],
                                 gsem.at[nb])

            return 0

        lax.fori_loop(0, C, chunk, 0)
        for t in range(C - 3, C):
            pltpu.make_async_copy(rows.at[t % 3], acc.at[didx.at[t]],
                                  ssem.at[t % 3]).wait()
        plsc.subcore_barrier()
        pltpu.sync_copy(acc.at[pl.ds(row0, rpt)], buf)
        pltpu.sync_copy(buf, out_hbm.at[c, pl.ds(row0, rpt)])

    return agg_kernel(hs, edge3)


def _tc_first(x, W, degp, block=1000):
    N, D = x.shape
    H = W.shape[1]

    def body(x_ref, w_ref, deg_ref, out_ref):
        deg = deg_ref[0, :, 0] + deg_ref[1, :, 0] + 1.0
        dinv = lax.rsqrt(deg)
        prod = jnp.dot(
            x_ref[...], w_ref[...],
            preferred_element_type=jnp.float32,
            precision=lax.Precision.HIGHEST,
        )
        out_ref[...] = prod * dinv[:, None]

    return pl.pallas_call(
        body,
        grid=(N // block,),
        in_specs=[
            pl.BlockSpec((block, D), lambda i: (i, 0)),
            pl.BlockSpec((D, H), lambda i: (0, 0)),
            pl.BlockSpec((2, block, 16), lambda i: (0, i, 0)),
        ],
        out_specs=pl.BlockSpec((block, H), lambda i: (i, 0)),
        out_shape=jax.ShapeDtypeStruct((N, H), jnp.float32),
    )(x, W, degp)


def _tc_layer(part, hs, degp, b, W, scale_out, block=1000):
    N, H = hs.shape
    Ho = W.shape[1]

    def body(part_ref, hs_ref, deg_ref, b_ref, w_ref, out_ref):
        deg = deg_ref[0, :, 0] + deg_ref[1, :, 0] + 1.0
        dinv = lax.rsqrt(deg)
        agg = part_ref[0] + part_ref[1] - hs_ref[...]
        h = jnp.maximum(agg * dinv[:, None] + b_ref[0, :][None, :], 0.0)
        prod = jnp.dot(
            h, w_ref[...],
            preferred_element_type=jnp.float32,
            precision=lax.Precision.HIGHEST,
        )
        if scale_out:
            out_ref[...] = prod * dinv[:, None]
        else:
            out_ref[...] = prod

    return pl.pallas_call(
        body,
        grid=(N // block,),
        in_specs=[
            pl.BlockSpec((2, block, H), lambda i: (0, i, 0)),
            pl.BlockSpec((block, H), lambda i: (i, 0)),
            pl.BlockSpec((2, block, 16), lambda i: (0, i, 0)),
            pl.BlockSpec((1, H), lambda i: (0, 0)),
            pl.BlockSpec((H, Ho), lambda i: (0, 0)),
        ],
        out_specs=pl.BlockSpec((block, Ho), lambda i: (i, 0)),
        out_shape=jax.ShapeDtypeStruct((N, Ho), jnp.float32),
    )(part, hs, degp, b, W)


def _tc_final(part, hs, degp, b, Wfc, bfc, block=1000):
    N, H = hs.shape
    C0 = Wfc.shape[1]
    CP = 128
    Wp = jnp.pad(Wfc, ((0, 0), (0, CP - C0)))
    bp = jnp.pad(bfc, (0, CP - C0))[None, :]

    def body(part_ref, hs_ref, deg_ref, b_ref, w_ref, bfc_ref, out_ref):
        deg = deg_ref[0, :, 0] + deg_ref[1, :, 0] + 1.0
        dinv = lax.rsqrt(deg)
        agg = part_ref[0] + part_ref[1] - hs_ref[...]
        h = jnp.maximum(agg * dinv[:, None] + b_ref[0, :][None, :], 0.0)
        prod = jnp.dot(
            h, w_ref[...],
            preferred_element_type=jnp.float32,
            precision=lax.Precision.HIGHEST,
        )
        out_ref[...] = prod + bfc_ref[0, :][None, :]

    outp = pl.pallas_call(
        body,
        grid=(N // block,),
        in_specs=[
            pl.BlockSpec((2, block, H), lambda i: (0, i, 0)),
            pl.BlockSpec((block, H), lambda i: (i, 0)),
            pl.BlockSpec((2, block, 16), lambda i: (0, i, 0)),
            pl.BlockSpec((1, H), lambda i: (0, 0)),
            pl.BlockSpec((H, CP), lambda i: (0, 0)),
            pl.BlockSpec((1, CP), lambda i: (0, 0)),
        ],
        out_specs=pl.BlockSpec((block, CP), lambda i: (i, 0)),
        out_shape=jax.ShapeDtypeStruct((N, CP), jnp.float32),
    )(part, hs, degp, b, Wp, bp)
    return outp[:, :C0]


@jax.jit
def kernel(x, edge_index, W1, b1, W2, b2, Wfc, bfc):
    N = x.shape[0]
    E = edge_index.shape[1]

    rpt = -(-N // (NS * 8)) * 8
    N2 = rpt * NS
    xp = jnp.pad(x, ((0, N2 - N), (0, 0)))
    block = N2

    K = 112
    per_w = -(-E // (NW * K)) * K
    E2 = per_w * NW
    C = per_w // K
    ei = edge_index.astype(jnp.int32)
    if E2 == E:
        edge3 = ei.reshape(2, NW, C, K)
    else:
        pad_src = jnp.full((1, E2 - E), N2 - 1, jnp.int32)
        pad_dst = N + (jnp.arange(E2 - E, dtype=jnp.int32) % (N2 - N))[None]
        edge3 = jnp.concatenate(
            [ei, jnp.concatenate([pad_src, pad_dst], 0)], 1
        ).reshape(2, NW, C, K)

    degp = _sc_degree(edge3, N2)
    hs1 = _tc_first(xp, W1, degp, block=block)
    part1 = _sc_aggregate(hs1, edge3)
    hs2 = _tc_layer(part1, hs1, degp, b1[None, :], W2, scale_out=True,
                    block=block)
    part2 = _sc_aggregate(hs2, edge3)
    out = _tc_final(part2, hs2, degp, b2[None, :], Wfc, bfc, block=block)
    return out[:N]


# speedup vs baseline: 1.4048x; 1.4048x over previous
import functools

import jax
import jax.numpy as jnp
from jax import lax
from jax.experimental import pallas as pl
from jax.experimental.pallas import tpu as pltpu
from jax.experimental.pallas import tpu_sc as plsc

NC = 2
NS = 16
NW = NC * NS


def _pick_chunk(per_worker):
    for k in range(128, 7, -8):
        if per_worker % k == 0:
            return k
    return 0


def _sc_degree(edge3, n_nodes):
    _, _, C, K = edge3.shape
    rpt = n_nodes // NS
    mesh = plsc.VectorSubcoreMesh(core_axis_name="c", subcore_axis_name="s")

    @functools.partial(
        pl.kernel,
        out_type=jax.ShapeDtypeStruct((NC, n_nodes, 16), jnp.float32),
        mesh=mesh,
        compiler_params=pltpu.CompilerParams(use_tc_tiling_on_sc=False),
        scratch_types=[
            pltpu.VMEM((C, K), jnp.int32),
            pltpu.VMEM((K, 16), jnp.float32),
            pltpu.VMEM((rpt, 16), jnp.float32),
            pltpu.VMEM_SHARED((n_nodes, 16), jnp.float32),
            pltpu.SemaphoreType.DMA,
        ],
    )
    def deg_kernel(edge_hbm, out_hbm, idx_v, ones_v, buf_v, acc_sh, sem):
        c = lax.axis_index("c")
        s = lax.axis_index("s")
        wid = c * NS + s
        pltpu.sync_copy(edge_hbm.at[1, wid], idx_v)

        one_row = jnp.where(lax.iota(jnp.int32, 16) == 0, 1.0, 0.0)

        def fill_ones(i, _):
            ones_v[i, :] = one_row
            return 0

        lax.fori_loop(0, K, fill_ones, 0)

        zv = jnp.zeros((16,), jnp.float32)

        def fill_zero(i, _):
            buf_v[i, :] = zv
            return 0

        lax.fori_loop(0, rpt, fill_zero, 0)
        row0 = s * rpt
        pltpu.sync_copy(buf_v, acc_sh.at[pl.ds(row0, rpt)])
        plsc.subcore_barrier()

        pltpu.async_copy(ones_v, acc_sh.at[idx_v.at[0]], sem, add=True)

        def chunk(j, _):
            @pl.when(j + 1 < C)
            def _():
                pltpu.async_copy(ones_v, acc_sh.at[idx_v.at[j + 1]], sem,
                                 add=True)

            pltpu.make_async_copy(ones_v, acc_sh.at[idx_v.at[j]], sem).wait()
            return 0

        lax.fori_loop(0, C, chunk, 0)
        plsc.subcore_barrier()
        pltpu.sync_copy(acc_sh.at[pl.ds(row0, rpt)], buf_v)
        pltpu.sync_copy(buf_v, out_hbm.at[c, pl.ds(row0, rpt)])

    return deg_kernel(edge3)


def _sc_aggregate(hs, edge3):
    N, H = hs.shape
    _, _, C, K = edge3.shape
    rpt = N // NS
    mesh = plsc.VectorSubcoreMesh(core_axis_name="c", subcore_axis_name="s")

    @functools.partial(
        pl.kernel,
        out_type=jax.ShapeDtypeStruct((NC, N, H), jnp.float32),
        mesh=mesh,
        compiler_params=pltpu.CompilerParams(use_tc_tiling_on_sc=False),
        scratch_types=[
            pltpu.VMEM((C, K), jnp.int32),
            pltpu.VMEM((C, K), jnp.int32),
            pltpu.VMEM((3, K, H), jnp.float32),
            pltpu.VMEM((rpt, H), jnp.float32),
            pltpu.VMEM_SHARED((N, H), jnp.float32),
            pltpu.SemaphoreType.DMA((3,)),
            pltpu.SemaphoreType.DMA((3,)),
        ],
    )
    def agg_kernel(hs_hbm, edge_hbm, out_hbm, sidx, didx, rows, buf,
                   acc, gsem, ssem):
        c = lax.axis_index("c")
        s = lax.axis_index("s")
        wid = c * NS + s
        pltpu.sync_copy(edge_hbm.at[0, wid], sidx)
        pltpu.sync_copy(edge_hbm.at[1, wid], didx)

        row0 = s * rpt
        pltpu.sync_copy(hs_hbm.at[pl.ds(row0, rpt)], buf)
        pltpu.sync_copy(buf, acc.at[pl.ds(row0, rpt)])
        plsc.subcore_barrier()

        for p in range(2):
            pltpu.async_copy(hs_hbm.at[sidx.at[p]], rows.at[p], gsem.at[p])

        def chunk(j, _):
            b = lax.rem(j, 3)
            pltpu.make_async_copy(hs_hbm.at[sidx.at[j]], rows.at[b],
                                  gsem.at[b]).wait()
            pltpu.async_copy(rows.at[b], acc.at[didx.at[j]], ssem.at[b],
                             add=True)
            nb = lax.rem(j + 2, 3)

            @pl.when(j + 2 < C)
            def _():
                @pl.when(j >= 1)
                def _():
                    pltpu.make_async_copy(rows.at[nb], acc.at[didx.at[j - 1]],
                                          ssem.at[nb]).wait()

                pltpu.async_copy(hs_hbm.at[sidx.at[j + 2]], rows.at[nb],
                                 gsem.at[nb])

            return 0

        lax.fori_loop(0, C, chunk, 0)
        for t in range(C - 3, C):
            pltpu.make_async_copy(rows.at[t % 3], acc.at[didx.at[t]],
                                  ssem.at[t % 3]).wait()
        plsc.subcore_barrier()
        pltpu.sync_copy(acc.at[pl.ds(row0, rpt)], buf)
        pltpu.sync_copy(buf, out_hbm.at[c, pl.ds(row0, rpt)])

    return agg_kernel(hs, edge3)


def _tc_first(x, W, degp, block=1000):
    N, D = x.shape
    H = W.shape[1]

    def body(x_ref, w_ref, deg_ref, out_ref):
        deg = deg_ref[0, :, 0] + deg_ref[1, :, 0] + 1.0
        dinv = lax.rsqrt(deg)
        prod = jnp.dot(
            x_ref[...], w_ref[...],
            preferred_element_type=jnp.float32,
            precision=lax.Precision.HIGHEST,
        )
        out_ref[...] = prod * dinv[:, None]

    return pl.pallas_call(
        body,
        grid=(N // block,),
        in_specs=[
            pl.BlockSpec((block, D), lambda i: (i, 0)),
            pl.BlockSpec((D, H), lambda i: (0, 0)),
            pl.BlockSpec((2, block, 16), lambda i: (0, i, 0)),
        ],
        out_specs=pl.BlockSpec((block, H), lambda i: (i, 0)),
        out_shape=jax.ShapeDtypeStruct((N, H), jnp.float32),
    )(x, W, degp)


def _tc_layer(part, hs, degp, b, W, scale_out, block=1000):
    N, H = hs.shape
    Ho = W.shape[1]

    def body(part_ref, hs_ref, deg_ref, b_ref, w_ref, out_ref):
        deg = deg_ref[0, :, 0] + deg_ref[1, :, 0] + 1.0
        dinv = lax.rsqrt(deg)
        agg = part_ref[0] + part_ref[1] - hs_ref[...]
        h = jnp.maximum(agg * dinv[:, None] + b_ref[0, :][None, :], 0.0)
        prod = jnp.dot(
            h, w_ref[...],
            preferred_element_type=jnp.float32,
            precision=lax.Precision.HIGHEST,
        )
        if scale_out:
            out_ref[...] = prod * dinv[:, None]
        else:
            out_ref[...] = prod

    return pl.pallas_call(
        body,
        grid=(N // block,),
        in_specs=[
            pl.BlockSpec((2, block, H), lambda i: (0, i, 0)),
            pl.BlockSpec((block, H), lambda i: (i, 0)),
            pl.BlockSpec((2, block, 16), lambda i: (0, i, 0)),
            pl.BlockSpec((1, H), lambda i: (0, 0)),
            pl.BlockSpec((H, Ho), lambda i: (0, 0)),
        ],
        out_specs=pl.BlockSpec((block, Ho), lambda i: (i, 0)),
        out_shape=jax.ShapeDtypeStruct((N, Ho), jnp.float32),
    )(part, hs, degp, b, W)


def _tc_final(part, hs, degp, b, Wfc, bfc, block=1000):
    N, H = hs.shape
    C0 = Wfc.shape[1]
    CP = 128
    Wp = jnp.pad(Wfc, ((0, 0), (0, CP - C0)))
    bp = jnp.pad(bfc, (0, CP - C0))[None, :]

    def body(part_ref, hs_ref, deg_ref, b_ref, w_ref, bfc_ref, out_ref):
        deg = deg_ref[0, :, 0] + deg_ref[1, :, 0] + 1.0
        dinv = lax.rsqrt(deg)
        agg = part_ref[0] + part_ref[1] - hs_ref[...]
        h = jnp.maximum(agg * dinv[:, None] + b_ref[0, :][None, :], 0.0)
        prod = jnp.dot(
            h, w_ref[...],
            preferred_element_type=jnp.float32,
            precision=lax.Precision.HIGHEST,
        )
        out_ref[...] = prod + bfc_ref[0, :][None, :]

    outp = pl.pallas_call(
        body,
        grid=(N // block,),
        in_specs=[
            pl.BlockSpec((2, block, H), lambda i: (0, i, 0)),
            pl.BlockSpec((block, H), lambda i: (i, 0)),
            pl.BlockSpec((2, block, 16), lambda i: (0, i, 0)),
            pl.BlockSpec((1, H), lambda i: (0, 0)),
            pl.BlockSpec((H, CP), lambda i: (0, 0)),
            pl.BlockSpec((1, CP), lambda i: (0, 0)),
        ],
        out_specs=pl.BlockSpec((block, CP), lambda i: (i, 0)),
        out_shape=jax.ShapeDtypeStruct((N, CP), jnp.float32),
    )(part, hs, degp, b, Wp, bp)
    return outp[:, :C0]


@jax.jit
def kernel(x, edge_index, W1, b1, W2, b2, Wfc, bfc):
    N = x.shape[0]
    E = edge_index.shape[1]

    rpt = -(-N // (NS * 8)) * 8
    N2 = rpt * NS
    xp = jnp.pad(x, ((0, N2 - N), (0, 0)))
    block = N2

    K = 80
    per_w = -(-E // (NW * K)) * K
    E2 = per_w * NW
    C = per_w // K
    ei = edge_index.astype(jnp.int32)
    if E2 == E:
        edge3 = ei.reshape(2, NW, C, K)
    else:
        pad_src = jnp.full((1, E2 - E), N2 - 1, jnp.int32)
        pad_dst = N + (jnp.arange(E2 - E, dtype=jnp.int32) % (N2 - N))[None]
        edge3 = jnp.concatenate(
            [ei, jnp.concatenate([pad_src, pad_dst], 0)], 1
        ).reshape(2, NW, C, K)

    degp = _sc_degree(edge3, N2)
    hs1 = _tc_first(xp, W1, degp, block=block)
    part1 = _sc_aggregate(hs1, edge3)
    hs2 = _tc_layer(part1, hs1, degp, b1[None, :], W2, scale_out=True,
                    block=block)
    part2 = _sc_aggregate(hs2, edge3)
    out = _tc_final(part2, hs2, degp, b2[None, :], Wfc, bfc, block=block)
    return out[:N]
